# baseline (speedup 1.0000x reference)
import functools

import jax
import jax.numpy as jnp
import numpy as np
from jax import lax
from jax.experimental import pallas as pl
from jax.experimental.pallas import tpu as pltpu
from jax.experimental.pallas import tpu_sc as plsc

N = 10000
E = 320000
D = 128
NSTEPS = 4
NC, NS, L = 2, 16, 16
NW = NC * NS
EPT = E // NW
K = 80
EPT2 = 10240
NCH = EPT2 // K
NP = 10112
RPT = NP // NS
NDISC = NP - N

_MESH = plsc.VectorSubcoreMesh(core_axis_name="c", subcore_axis_name="s",
                               num_cores=NC, num_subcores=NS)
_SC_PARAMS = pltpu.CompilerParams(needs_layout_passes=False)


def _wid():
    return lax.axis_index("c") * NS + lax.axis_index("s")


def _sc_deg_body(dst_hbm, out_hbm, dst_v, acc_v):
    w = _wid()
    pltpu.sync_copy(dst_hbm.at[pl.ds(w * EPT, EPT)], dst_v)

    def zero(j, carry):
        acc_v[pl.ds(j * L, L)] = jnp.zeros((L,), jnp.float32)
        return carry

    lax.fori_loop(0, N // L, zero, 0)
    ones = jnp.full((L,), 1.0, jnp.float32)

    def body(j, carry):
        didx = dst_v[pl.ds(j * L, L)]
        plsc.addupdate_scatter(acc_v, [didx], ones)
        return carry

    lax.fori_loop(0, EPT // L, body, 0)
    pltpu.sync_copy(acc_v, out_hbm.at[w, 0])


_sc_deg = pl.kernel(
    _sc_deg_body,
    out_type=jax.ShapeDtypeStruct((NW, 1, N), jnp.float32),
    mesh=_MESH,
    scratch_types=[
        pltpu.VMEM((EPT,), jnp.int32),
        pltpu.VMEM((N,), jnp.float32),
    ],
    compiler_params=_SC_PARAMS,
)


def _sc_g_body(src_hbm, dst_hbm, dinv_hbm, out_hbm, src_v, dst_v, dinv_v, acc_v):
    w = _wid()
    pltpu.sync_copy(src_hbm.at[pl.ds(w * EPT, EPT)], src_v)
    pltpu.sync_copy(dst_hbm.at[pl.ds(w * EPT, EPT)], dst_v)
    pltpu.sync_copy(dinv_hbm, dinv_v)

    def zero(j, carry):
        acc_v[pl.ds(j * L, L)] = jnp.zeros((L,), jnp.float32)
        return carry

    lax.fori_loop(0, N // L, zero, 0)

    def body(j, carry):
        sidx = src_v[pl.ds(j * L, L)]
        didx = dst_v[pl.ds(j * L, L)]
        vals = plsc.load_gather(dinv_v, [sidx])
        plsc.addupdate_scatter(acc_v, [didx], vals)
        return carry

    lax.fori_loop(0, EPT // L, body, 0)
    pltpu.sync_copy(acc_v, out_hbm.at[w, 0])


_sc_g = pl.kernel(
    _sc_g_body,
    out_type=jax.ShapeDtypeStruct((NW, 1, N), jnp.float32),
    mesh=_MESH,
    scratch_types=[
        pltpu.VMEM((EPT,), jnp.int32),
        pltpu.VMEM((EPT,), jnp.int32),
        pltpu.VMEM((N,), jnp.float32),
        pltpu.VMEM((N,), jnp.float32),
    ],
    compiler_params=_SC_PARAMS,
)


NBUF = 5
GLAG = 2


def _sc_agg_body(src2_hbm, dst2_hbm, yp_hbm, zrows_hbm, out_hbm,
                 src_v, dst_v, rows_a, acc_sh, sem_a):
    c = lax.axis_index("c")
    s = lax.axis_index("s")
    w = c * NS + s
    pltpu.sync_copy(src2_hbm.at[w], src_v)
    pltpu.sync_copy(dst2_hbm.at[w], dst_v)
    pltpu.sync_copy(zrows_hbm, acc_sh.at[pl.ds(s * RPT, RPT)])
    plsc.subcore_barrier()

    def body(j, carry):
        pltpu.async_copy(yp_hbm.at[src_v.at[j]], rows_a, sem_a).wait()
        pltpu.sync_copy(rows_a, acc_sh.at[dst_v.at[j]], add=True)
        return carry

    lax.fori_loop(0, NCH, body, 0)
    plsc.subcore_barrier()
    pltpu.sync_copy(acc_sh.at[pl.ds(s * RPT, RPT)], out_hbm.at[c, pl.ds(s * RPT, RPT)])


_sc_agg = pl.kernel(
    _sc_agg_body,
    out_type=jax.ShapeDtypeStruct((NC, NP, D), jnp.float32),
    mesh=_MESH,
    scratch_types=[
        pltpu.VMEM((NCH, K), jnp.int32),
        pltpu.VMEM((NCH, K), jnp.int32),
        pltpu.VMEM((K, D), jnp.float32),
        pltpu.VMEM_SHARED((NP, D), jnp.float32),
        pltpu.SemaphoreType.DMA,
    ],
    compiler_params=_SC_PARAMS,
)


def _tc_emb_body(x_ref, ew_ref, eb_ref, pdeg_ref, h_ref, hp_ref, dinv_ref, dsq_ref):
    h = jnp.dot(x_ref[...], ew_ref[...], preferred_element_type=jnp.float32)
    h = h + eb_ref[...]
    deg = jnp.sum(pdeg_ref[...], axis=0) + 1.0
    dinv = lax.rsqrt(deg)
    h_ref[...] = h
    hp_ref[...] = h * dinv[:, None]
    dinv_ref[...] = dinv
    dsq_ref[...] = 1.0 / deg


_tc_emb = pl.pallas_call(
    _tc_emb_body,
    out_shape=[
        jax.ShapeDtypeStruct((N, D), jnp.float32),
        jax.ShapeDtypeStruct((N, D), jnp.float32),
        jax.ShapeDtypeStruct((N,), jnp.float32),
        jax.ShapeDtypeStruct((N,), jnp.float32),
    ],
)


def _tc_s_body(g_ref, dinv_ref, dsq_ref, s_ref):
    s_ref[...] = dinv_ref[...] * jnp.sum(g_ref[...], axis=0) + dsq_ref[...]


_tc_s = pl.pallas_call(
    _tc_s_body,
    out_shape=jax.ShapeDtypeStruct((N,), jnp.float32),
)


def _tc_step_body(t, dt, p_ref, y_ref, dinv_ref, dsq_ref, s_ref,
                  w1_ref, w0_ref, b_ref, ynew_ref, ypnew_ref):
    dinv = dinv_ref[...]
    y = y_ref[...]
    p = p_ref[0, :N] + p_ref[1, :N]
    agg = dinv[:, None] * p + dsq_ref[...][:, None] * y
    conv = jnp.dot(agg, w1_ref[...], preferred_element_type=jnp.float32)
    conv = conv + (t * s_ref[...])[:, None] * w0_ref[...][None, :] + b_ref[...]
    ynew = y + dt * conv
    ynew_ref[...] = ynew
    ypnew_ref[...] = ynew * dinv[:, None]


def _make_tc_step(t, dt):
    return pl.pallas_call(
        functools.partial(_tc_step_body, t, dt),
        out_shape=[
            jax.ShapeDtypeStruct((N, D), jnp.float32),
            jax.ShapeDtypeStruct((N, D), jnp.float32),
        ],
    )


_TS = np.linspace(0.0, 1.0, NSTEPS)
_TC_STEPS = [_make_tc_step(float(_TS[i - 1]), float(_TS[i] - _TS[i - 1]))
             for i in range(1, NSTEPS)]


def kernel(x, edge_index, emb_W, emb_b, gcn_W, gcn_b):
    src = edge_index[0].astype(jnp.int32)
    dst = edge_index[1].astype(jnp.int32)
    srcr = src.reshape(NW, EPT)
    dstr = dst.reshape(NW, EPT)
    fsrc = jnp.zeros((NW, EPT2 - EPT), jnp.int32)
    fdst = jnp.broadcast_to(
        N + (jnp.arange(EPT2 - EPT, dtype=jnp.int32) % NDISC), (NW, EPT2 - EPT))
    src3 = jnp.concatenate([srcr, fsrc], axis=1).reshape(NW, NCH, K)
    dst3 = jnp.concatenate([dstr, fdst], axis=1).reshape(NW, NCH, K)
    zrows = jnp.zeros((RPT, D), jnp.float32)

    pdeg = _sc_deg(dst).reshape(NW, N)
    h, hp, dinv, dsq = _tc_emb(x, emb_W, emb_b, pdeg)
    g = _sc_g(src, dst, dinv).reshape(NW, N)
    s = _tc_s(g, dinv, dsq)

    w0 = gcn_W[0]
    w1 = gcn_W[1:]
    outs = [h]
    y, yp = h, hp
    for i in range(1, NSTEPS):
        p = _sc_agg(src3, dst3, yp, zrows)
        y, yp = _TC_STEPS[i - 1](p, y, dinv, dsq, s, w1, w0, gcn_b)
        outs.append(y)
    return jnp.stack(outs, axis=0)


# speedup vs baseline: 2.7709x; 2.7709x over previous
import functools

import jax
import jax.numpy as jnp
import numpy as np
from jax import lax
from jax.experimental import pallas as pl
from jax.experimental.pallas import tpu as pltpu
from jax.experimental.pallas import tpu_sc as plsc

N = 10000
E = 320000
D = 128
NSTEPS = 4
NC, NS, L = 2, 16, 16
NW = NC * NS
EPT = E // NW
K = 125
NCH = EPT // K
NP = 10112
RPT = NP // NS

_MESH = plsc.VectorSubcoreMesh(core_axis_name="c", subcore_axis_name="s",
                               num_cores=NC, num_subcores=NS)
_SC_PARAMS = pltpu.CompilerParams(needs_layout_passes=False)


def _wid():
    return lax.axis_index("c") * NS + lax.axis_index("s")


def _sc_deg_body(dst_hbm, out_hbm, dst_v, acc_v):
    w = _wid()
    pltpu.sync_copy(dst_hbm.at[pl.ds(w * EPT, EPT)], dst_v)

    def zero(j, carry):
        acc_v[pl.ds(j * L, L)] = jnp.zeros((L,), jnp.float32)
        return carry

    lax.fori_loop(0, N // L, zero, 0)
    ones = jnp.full((L,), 1.0, jnp.float32)

    def body(j, carry):
        didx = dst_v[pl.ds(j * L, L)]
        plsc.addupdate_scatter(acc_v, [didx], ones)
        return carry

    lax.fori_loop(0, EPT // L, body, 0)
    pltpu.sync_copy(acc_v, out_hbm.at[w, 0])


_sc_deg = pl.kernel(
    _sc_deg_body,
    out_type=jax.ShapeDtypeStruct((NW, 1, N), jnp.float32),
    mesh=_MESH,
    scratch_types=[
        pltpu.VMEM((EPT,), jnp.int32),
        pltpu.VMEM((N,), jnp.float32),
    ],
    compiler_params=_SC_PARAMS,
)


def _sc_g_body(src_hbm, dst_hbm, dinv_hbm, out_hbm, src_v, dst_v, dinv_v, acc_v):
    w = _wid()
    pltpu.sync_copy(src_hbm.at[pl.ds(w * EPT, EPT)], src_v)
    pltpu.sync_copy(dst_hbm.at[pl.ds(w * EPT, EPT)], dst_v)
    pltpu.sync_copy(dinv_hbm, dinv_v)

    def zero(j, carry):
        acc_v[pl.ds(j * L, L)] = jnp.zeros((L,), jnp.float32)
        return carry

    lax.fori_loop(0, N // L, zero, 0)

    def body(j, carry):
        sidx = src_v[pl.ds(j * L, L)]
        didx = dst_v[pl.ds(j * L, L)]
        vals = plsc.load_gather(dinv_v, [sidx])
        plsc.addupdate_scatter(acc_v, [didx], vals)
        return carry

    lax.fori_loop(0, EPT // L, body, 0)
    pltpu.sync_copy(acc_v, out_hbm.at[w, 0])


_sc_g = pl.kernel(
    _sc_g_body,
    out_type=jax.ShapeDtypeStruct((NW, 1, N), jnp.float32),
    mesh=_MESH,
    scratch_types=[
        pltpu.VMEM((EPT,), jnp.int32),
        pltpu.VMEM((EPT,), jnp.int32),
        pltpu.VMEM((N,), jnp.float32),
        pltpu.VMEM((N,), jnp.float32),
    ],
    compiler_params=_SC_PARAMS,
)


NBUF = 5
GLAG = 2


def _sc_agg_body(pk_hbm, yp_hbm, zrows_hbm, out_hbm,
                 pk_v, rows_a, rows_b, sidx_a, didx_a, sidx_b, didx_b,
                 acc_sh, sem_a, sem_b):
    c = lax.axis_index("c")
    s = lax.axis_index("s")
    w = c * NS + s
    pltpu.sync_copy(pk_hbm.at[w], pk_v)
    pltpu.sync_copy(zrows_hbm, acc_sh.at[pl.ds(s * RPT, RPT)])
    plsc.subcore_barrier()

    def unpack(j, sidx_st, didx_st):
        for off in list(range(0, K - L, L)) + [K - L]:
            v = pk_v[j, pl.ds(off, L)]
            sidx_st[0, pl.ds(off, L)] = jnp.right_shift(v, 14)
            didx_st[0, pl.ds(off, L)] = jnp.bitwise_and(v, 16383)

    def body(m, carry):
        j0 = 2 * m
        j1 = 2 * m + 1
        unpack(j0, sidx_a, didx_a)
        da = pltpu.async_copy(yp_hbm.at[sidx_a.at[0]], rows_a, sem_a)
        unpack(j1, sidx_b, didx_b)
        db = pltpu.async_copy(yp_hbm.at[sidx_b.at[0]], rows_b, sem_b)
        da.wait()
        pltpu.sync_copy(rows_a, acc_sh.at[didx_a.at[0]], add=True)
        db.wait()
        pltpu.sync_copy(rows_b, acc_sh.at[didx_b.at[0]], add=True)
        return carry

    lax.fori_loop(0, NCH // 2, body, 0)
    plsc.subcore_barrier()
    pltpu.sync_copy(acc_sh.at[pl.ds(s * RPT, RPT)], out_hbm.at[c, pl.ds(s * RPT, RPT)])


_sc_agg = pl.kernel(
    _sc_agg_body,
    out_type=jax.ShapeDtypeStruct((NC, NP, D), jnp.float32),
    mesh=_MESH,
    scratch_types=[
        pltpu.VMEM((NCH, K), jnp.int32),
        pltpu.VMEM((K, D), jnp.float32),
        pltpu.VMEM((K, D), jnp.float32),
        pltpu.VMEM((1, K), jnp.int32),
        pltpu.VMEM((1, K), jnp.int32),
        pltpu.VMEM((1, K), jnp.int32),
        pltpu.VMEM((1, K), jnp.int32),
        pltpu.VMEM_SHARED((NP, D), jnp.float32),
        pltpu.SemaphoreType.DMA,
        pltpu.SemaphoreType.DMA,
    ],
    compiler_params=_SC_PARAMS,
)


def _tc_emb_body(x_ref, ew_ref, eb_ref, pdeg_ref, h_ref, hp_ref, dinv_ref, dsq_ref):
    h = jnp.dot(x_ref[...], ew_ref[...], preferred_element_type=jnp.float32)
    h = h + eb_ref[...]
    deg = jnp.sum(pdeg_ref[...], axis=0) + 1.0
    dinv = lax.rsqrt(deg)
    h_ref[...] = h
    hp_ref[...] = h * dinv[:, None]
    dinv_ref[...] = dinv
    dsq_ref[...] = 1.0 / deg


_tc_emb = pl.pallas_call(
    _tc_emb_body,
    out_shape=[
        jax.ShapeDtypeStruct((N, D), jnp.float32),
        jax.ShapeDtypeStruct((N, D), jnp.float32),
        jax.ShapeDtypeStruct((N,), jnp.float32),
        jax.ShapeDtypeStruct((N,), jnp.float32),
    ],
)


def _tc_s_body(g_ref, dinv_ref, dsq_ref, s_ref):
    s_ref[...] = dinv_ref[...] * jnp.sum(g_ref[...], axis=0) + dsq_ref[...]


_tc_s = pl.pallas_call(
    _tc_s_body,
    out_shape=jax.ShapeDtypeStruct((N,), jnp.float32),
)


def _tc_step_body(t, dt, p_ref, y_ref, dinv_ref, dsq_ref, s_ref,
                  w1_ref, w0_ref, b_ref, ynew_ref, ypnew_ref):
    dinv = dinv_ref[...]
    y = y_ref[...]
    p = p_ref[0, :N] + p_ref[1, :N]
    agg = dinv[:, None] * p + dsq_ref[...][:, None] * y
    conv = jnp.dot(agg, w1_ref[...], preferred_element_type=jnp.float32)
    conv = conv + (t * s_ref[...])[:, None] * w0_ref[...][None, :] + b_ref[...]
    ynew = y + dt * conv
    ynew_ref[...] = ynew
    ypnew_ref[...] = ynew * dinv[:, None]


def _make_tc_step(t, dt):
    return pl.pallas_call(
        functools.partial(_tc_step_body, t, dt),
        out_shape=[
            jax.ShapeDtypeStruct((N, D), jnp.float32),
            jax.ShapeDtypeStruct((N, D), jnp.float32),
        ],
    )


_TS = np.linspace(0.0, 1.0, NSTEPS)
_TC_STEPS = [_make_tc_step(float(_TS[i - 1]), float(_TS[i] - _TS[i - 1]))
             for i in range(1, NSTEPS)]


def kernel(x, edge_index, emb_W, emb_b, gcn_W, gcn_b):
    src = edge_index[0].astype(jnp.int32)
    dst = edge_index[1].astype(jnp.int32)
    packed = ((src << 14) | dst).reshape(NW, NCH, K)
    zrows = jnp.zeros((RPT, D), jnp.float32)

    pdeg = _sc_deg(dst).reshape(NW, N)
    h, hp, dinv, dsq = _tc_emb(x, emb_W, emb_b, pdeg)
    g = _sc_g(src, dst, dinv).reshape(NW, N)
    s = _tc_s(g, dinv, dsq)

    w0 = gcn_W[0]
    w1 = gcn_W[1:]
    outs = [h]
    y, yp = h, hp
    for i in range(1, NSTEPS):
        p = _sc_agg(packed, yp, zrows)
        y, yp = _TC_STEPS[i - 1](p, y, dinv, dsq, s, w1, w0, gcn_b)
        outs.append(y)
    return jnp.stack(outs, axis=0)


# speedup vs baseline: 3.0554x; 1.1027x over previous
import functools

import jax
import jax.numpy as jnp
import numpy as np
from jax import lax
from jax.experimental import pallas as pl
from jax.experimental.pallas import tpu as pltpu
from jax.experimental.pallas import tpu_sc as plsc

N = 10000
E = 320000
D = 128
NSTEPS = 4
NC, NS, L = 2, 16, 16
NW = NC * NS
EPT = E // NW
K = 125
NCH = EPT // K
NP = 10112
RPT = NP // NS

_MESH = plsc.VectorSubcoreMesh(core_axis_name="c", subcore_axis_name="s",
                               num_cores=NC, num_subcores=NS)
_SC_PARAMS = pltpu.CompilerParams(needs_layout_passes=False)


def _wid():
    return lax.axis_index("c") * NS + lax.axis_index("s")


def _sc_deg_body(dst_hbm, out_hbm, dst_v, acc_v):
    w = _wid()
    pltpu.sync_copy(dst_hbm.at[pl.ds(w * EPT, EPT)], dst_v)

    def zero(j, carry):
        acc_v[pl.ds(j * L, L)] = jnp.zeros((L,), jnp.float32)
        return carry

    lax.fori_loop(0, N // L, zero, 0)
    ones = jnp.full((L,), 1.0, jnp.float32)

    def body(j, carry):
        didx = dst_v[pl.ds(j * L, L)]
        plsc.addupdate_scatter(acc_v, [didx], ones)
        return carry

    lax.fori_loop(0, EPT // L, body, 0)
    pltpu.sync_copy(acc_v, out_hbm.at[w, 0])


_sc_deg = pl.kernel(
    _sc_deg_body,
    out_type=jax.ShapeDtypeStruct((NW, 1, N), jnp.float32),
    mesh=_MESH,
    scratch_types=[
        pltpu.VMEM((EPT,), jnp.int32),
        pltpu.VMEM((N,), jnp.float32),
    ],
    compiler_params=_SC_PARAMS,
)


def _sc_g_body(src_hbm, dst_hbm, dinv_hbm, out_hbm, src_v, dst_v, dinv_v, acc_v):
    w = _wid()
    pltpu.sync_copy(src_hbm.at[pl.ds(w * EPT, EPT)], src_v)
    pltpu.sync_copy(dst_hbm.at[pl.ds(w * EPT, EPT)], dst_v)
    pltpu.sync_copy(dinv_hbm, dinv_v)

    def zero(j, carry):
        acc_v[pl.ds(j * L, L)] = jnp.zeros((L,), jnp.float32)
        return carry

    lax.fori_loop(0, N // L, zero, 0)

    def body(j, carry):
        sidx = src_v[pl.ds(j * L, L)]
        didx = dst_v[pl.ds(j * L, L)]
        vals = plsc.load_gather(dinv_v, [sidx])
        plsc.addupdate_scatter(acc_v, [didx], vals)
        return carry

    lax.fori_loop(0, EPT // L, body, 0)
    pltpu.sync_copy(acc_v, out_hbm.at[w, 0])


_sc_g = pl.kernel(
    _sc_g_body,
    out_type=jax.ShapeDtypeStruct((NW, 1, N), jnp.float32),
    mesh=_MESH,
    scratch_types=[
        pltpu.VMEM((EPT,), jnp.int32),
        pltpu.VMEM((EPT,), jnp.int32),
        pltpu.VMEM((N,), jnp.float32),
        pltpu.VMEM((N,), jnp.float32),
    ],
    compiler_params=_SC_PARAMS,
)


NBUF = 5
GLAG = 2


def _sc_agg_body(pk_hbm, yp_hbm, zrows_hbm, out_hbm,
                 pk_v, rows_a, rows_b, sidx_a, didx_a, sidx_b, didx_b,
                 acc_sh, sem_a, sem_b, sem_sa, sem_sb):
    c = lax.axis_index("c")
    s = lax.axis_index("s")
    w = c * NS + s
    pltpu.sync_copy(pk_hbm.at[w], pk_v)
    pltpu.sync_copy(zrows_hbm, acc_sh.at[pl.ds(s * RPT, RPT)])
    plsc.subcore_barrier()

    def unpack(j, sidx_st, didx_st):
        for off in list(range(0, K - L, L)) + [K - L]:
            v = pk_v[j, pl.ds(off, L)]
            sidx_st[0, pl.ds(off, L)] = jnp.right_shift(v, 14)
            didx_st[0, pl.ds(off, L)] = jnp.bitwise_and(v, 16383)

    def wait_gather(sidx_st, rows_x, sem_x):
        pltpu.make_async_copy(yp_hbm.at[sidx_st.at[0]], rows_x, sem_x).wait()

    def wait_scatter(didx_st, rows_x, sem_x):
        pltpu.make_async_copy(rows_x, acc_sh.at[didx_st.at[0]], sem_x).wait()

    bufs = [(sidx_a, didx_a, rows_a, sem_a, sem_sa),
            (sidx_b, didx_b, rows_b, sem_b, sem_sb)]

    def slot(j, par, first):
        sx, dx, rx, gx, ssx = bufs[par]
        sy, dy, ry, gy, ssy = bufs[1 - par]

        def fwd_prev():
            wait_gather(sy, ry, gy)
            pltpu.async_copy(ry, acc_sh.at[dy.at[0]], ssy, add=True)

        def free_own():
            wait_scatter(dx, rx, ssx)

        if first:
            pl.when(j > 0)(fwd_prev)
            pl.when(j > 1)(free_own)
        else:
            fwd_prev()
            pl.when(j > 1)(free_own)
        unpack(j, sx, dx)
        pltpu.async_copy(yp_hbm.at[sx.at[0]], rx, gx)

    def body(m, carry):
        slot(2 * m, 0, True)
        slot(2 * m + 1, 1, False)
        return carry

    lax.fori_loop(0, NCH // 2, body, 0)
    sx, dx, rx, gx, ssx = bufs[(NCH - 1) % 2]
    sy, dy, ry, gy, ssy = bufs[NCH % 2]
    wait_gather(sx, rx, gx)
    pltpu.async_copy(rx, acc_sh.at[dx.at[0]], ssx, add=True)
    wait_scatter(dy, ry, ssy)
    wait_scatter(dx, rx, ssx)
    plsc.subcore_barrier()
    pltpu.sync_copy(acc_sh.at[pl.ds(s * RPT, RPT)], out_hbm.at[c, pl.ds(s * RPT, RPT)])


_sc_agg = pl.kernel(
    _sc_agg_body,
    out_type=jax.ShapeDtypeStruct((NC, NP, D), jnp.float32),
    mesh=_MESH,
    scratch_types=[
        pltpu.VMEM((NCH, K), jnp.int32),
        pltpu.VMEM((K, D), jnp.float32),
        pltpu.VMEM((K, D), jnp.float32),
        pltpu.VMEM((1, K), jnp.int32),
        pltpu.VMEM((1, K), jnp.int32),
        pltpu.VMEM((1, K), jnp.int32),
        pltpu.VMEM((1, K), jnp.int32),
        pltpu.VMEM_SHARED((NP, D), jnp.float32),
        pltpu.SemaphoreType.DMA,
        pltpu.SemaphoreType.DMA,
        pltpu.SemaphoreType.DMA,
        pltpu.SemaphoreType.DMA,
    ],
    compiler_params=_SC_PARAMS,
)


def _tc_emb_body(x_ref, ew_ref, eb_ref, pdeg_ref, h_ref, hp_ref, dinv_ref, dsq_ref):
    h = jnp.dot(x_ref[...], ew_ref[...], preferred_element_type=jnp.float32)
    h = h + eb_ref[...]
    deg = jnp.sum(pdeg_ref[...], axis=0) + 1.0
    dinv = lax.rsqrt(deg)
    h_ref[...] = h
    hp_ref[...] = h * dinv[:, None]
    dinv_ref[...] = dinv
    dsq_ref[...] = 1.0 / deg


_tc_emb = pl.pallas_call(
    _tc_emb_body,
    out_shape=[
        jax.ShapeDtypeStruct((N, D), jnp.float32),
        jax.ShapeDtypeStruct((N, D), jnp.float32),
        jax.ShapeDtypeStruct((N,), jnp.float32),
        jax.ShapeDtypeStruct((N,), jnp.float32),
    ],
)


def _tc_s_body(g_ref, dinv_ref, dsq_ref, s_ref):
    s_ref[...] = dinv_ref[...] * jnp.sum(g_ref[...], axis=0) + dsq_ref[...]


_tc_s = pl.pallas_call(
    _tc_s_body,
    out_shape=jax.ShapeDtypeStruct((N,), jnp.float32),
)


def _tc_step_body(t, dt, p_ref, y_ref, dinv_ref, dsq_ref, s_ref,
                  w1_ref, w0_ref, b_ref, ynew_ref, ypnew_ref):
    dinv = dinv_ref[...]
    y = y_ref[...]
    p = p_ref[0, :N] + p_ref[1, :N]
    agg = dinv[:, None] * p + dsq_ref[...][:, None] * y
    conv = jnp.dot(agg, w1_ref[...], preferred_element_type=jnp.float32)
    conv = conv + (t * s_ref[...])[:, None] * w0_ref[...][None, :] + b_ref[...]
    ynew = y + dt * conv
    ynew_ref[...] = ynew
    ypnew_ref[...] = ynew * dinv[:, None]


def _make_tc_step(t, dt):
    return pl.pallas_call(
        functools.partial(_tc_step_body, t, dt),
        out_shape=[
            jax.ShapeDtypeStruct((N, D), jnp.float32),
            jax.ShapeDtypeStruct((N, D), jnp.float32),
        ],
    )


_TS = np.linspace(0.0, 1.0, NSTEPS)
_TC_STEPS = [_make_tc_step(float(_TS[i - 1]), float(_TS[i] - _TS[i - 1]))
             for i in range(1, NSTEPS)]


def kernel(x, edge_index, emb_W, emb_b, gcn_W, gcn_b):
    src = edge_index[0].astype(jnp.int32)
    dst = edge_index[1].astype(jnp.int32)
    packed = ((src << 14) | dst).reshape(NW, NCH, K)
    zrows = jnp.zeros((RPT, D), jnp.float32)

    pdeg = _sc_deg(dst).reshape(NW, N)
    h, hp, dinv, dsq = _tc_emb(x, emb_W, emb_b, pdeg)
    g = _sc_g(src, dst, dinv).reshape(NW, N)
    s = _tc_s(g, dinv, dsq)

    w0 = gcn_W[0]
    w1 = gcn_W[1:]
    outs = [h]
    y, yp = h, hp
    for i in range(1, NSTEPS):
        p = _sc_agg(packed, yp, zrows)
        y, yp = _TC_STEPS[i - 1](p, y, dinv, dsq, s, w1, w0, gcn_b)
        outs.append(y)
    return jnp.stack(outs, axis=0)


# speedup vs baseline: 3.7449x; 1.2257x over previous
import functools

import jax
import jax.numpy as jnp
import numpy as np
from jax import lax
from jax.experimental import pallas as pl
from jax.experimental.pallas import tpu as pltpu
from jax.experimental.pallas import tpu_sc as plsc

N = 10000
E = 320000
D = 128
NSTEPS = 4
NC, NS, L = 2, 16, 16
NW = NC * NS
EPT = E // NW
K = 80
NCH = EPT // K
NP = 10112
RPT = NP // NS

_MESH = plsc.VectorSubcoreMesh(core_axis_name="c", subcore_axis_name="s",
                               num_cores=NC, num_subcores=NS)
_SC_PARAMS = pltpu.CompilerParams(needs_layout_passes=False)


def _wid():
    return lax.axis_index("c") * NS + lax.axis_index("s")


def _sc_deg_body(dst_hbm, out_hbm, dst_v, acc_v):
    w = _wid()
    pltpu.sync_copy(dst_hbm.at[pl.ds(w * EPT, EPT)], dst_v)

    def zero(j, carry):
        acc_v[pl.ds(j * L, L)] = jnp.zeros((L,), jnp.float32)
        return carry

    lax.fori_loop(0, N // L, zero, 0)
    ones = jnp.full((L,), 1.0, jnp.float32)

    def body(j, carry):
        didx = dst_v[pl.ds(j * L, L)]
        plsc.addupdate_scatter(acc_v, [didx], ones)
        return carry

    lax.fori_loop(0, EPT // L, body, 0)
    pltpu.sync_copy(acc_v, out_hbm.at[w, 0])


_sc_deg = pl.kernel(
    _sc_deg_body,
    out_type=jax.ShapeDtypeStruct((NW, 1, N), jnp.float32),
    mesh=_MESH,
    scratch_types=[
        pltpu.VMEM((EPT,), jnp.int32),
        pltpu.VMEM((N,), jnp.float32),
    ],
    compiler_params=_SC_PARAMS,
)


def _sc_g_body(src_hbm, dst_hbm, dinv_hbm, out_hbm, src_v, dst_v, dinv_v, acc_v):
    w = _wid()
    pltpu.sync_copy(src_hbm.at[pl.ds(w * EPT, EPT)], src_v)
    pltpu.sync_copy(dst_hbm.at[pl.ds(w * EPT, EPT)], dst_v)
    pltpu.sync_copy(dinv_hbm, dinv_v)

    def zero(j, carry):
        acc_v[pl.ds(j * L, L)] = jnp.zeros((L,), jnp.float32)
        return carry

    lax.fori_loop(0, N // L, zero, 0)

    def body(j, carry):
        sidx = src_v[pl.ds(j * L, L)]
        didx = dst_v[pl.ds(j * L, L)]
        vals = plsc.load_gather(dinv_v, [sidx])
        plsc.addupdate_scatter(acc_v, [didx], vals)
        return carry

    lax.fori_loop(0, EPT // L, body, 0)
    pltpu.sync_copy(acc_v, out_hbm.at[w, 0])


_sc_g = pl.kernel(
    _sc_g_body,
    out_type=jax.ShapeDtypeStruct((NW, 1, N), jnp.float32),
    mesh=_MESH,
    scratch_types=[
        pltpu.VMEM((EPT,), jnp.int32),
        pltpu.VMEM((EPT,), jnp.int32),
        pltpu.VMEM((N,), jnp.float32),
        pltpu.VMEM((N,), jnp.float32),
    ],
    compiler_params=_SC_PARAMS,
)


NBUF = 5
GLAG = 2


def _sc_agg_body(pk_hbm, yp_hbm, zrows_hbm, out_hbm,
                 pk_v, rows_a, rows_b, rows_c,
                 sidx_a, didx_a, sidx_b, didx_b, sidx_c, didx_c,
                 acc_sh, sem_ga, sem_gb, sem_gc, sem_sa, sem_sb, sem_sc):
    c = lax.axis_index("c")
    s = lax.axis_index("s")
    w = c * NS + s
    pltpu.sync_copy(pk_hbm.at[w, 0], pk_v)
    pltpu.sync_copy(zrows_hbm, acc_sh.at[pl.ds(s * RPT, RPT)])
    plsc.subcore_barrier()

    bufs = [(sidx_a, didx_a, rows_a, sem_ga, sem_sa),
            (sidx_b, didx_b, rows_b, sem_gb, sem_sb),
            (sidx_c, didx_c, rows_c, sem_gc, sem_sc)]

    def unpack(j, sidx_st, didx_st):
        for i in range(K // L):
            v = pk_v[pl.ds(j * K + i * L, L)]
            sidx_st[0, pl.ds(i * L, L)] = jnp.right_shift(v, 14)
            didx_st[0, pl.ds(i * L, L)] = jnp.bitwise_and(v, 16383)

    def start_gather(j, b):
        sx, dx, rx, gx, sx2 = bufs[b]
        unpack(j, sx, dx)
        pltpu.async_copy(yp_hbm.at[sx.at[0]], rx, gx)

    def fwd(b):
        sx, dx, rx, gx, sx2 = bufs[b]
        pltpu.make_async_copy(yp_hbm.at[sx.at[0]], rx, gx).wait()
        pltpu.async_copy(rx, acc_sh.at[dx.at[0]], sx2, add=True)

    def wait_scatter(b):
        sx, dx, rx, gx, sx2 = bufs[b]
        pltpu.make_async_copy(rx, acc_sh.at[dx.at[0]], sx2).wait()

    def body(m, carry):
        for r in range(3):
            j = 3 * m + r
            if r < 2:
                pl.when(m > 0)(lambda b=(r + 1) % 3: fwd(b))
                pl.when(m > 0)(lambda b=r: wait_scatter(b))
            else:
                fwd(0)
                pl.when(m > 0)(lambda: wait_scatter(2))
            start_gather(j, r)
        return carry

    lax.fori_loop(0, NCH // 3, body, 0)
    for j in (NCH - 2, NCH - 1):
        b = j % 3
        fwd((j - 2) % 3)
        wait_scatter(b)
        start_gather(j, b)
    fwd((NCH - 2) % 3)
    fwd((NCH - 1) % 3)
    for b in range(3):
        wait_scatter(b)
    plsc.subcore_barrier()
    pltpu.sync_copy(acc_sh.at[pl.ds(s * RPT, RPT)], out_hbm.at[c, pl.ds(s * RPT, RPT)])


_sc_agg = pl.kernel(
    _sc_agg_body,
    out_type=jax.ShapeDtypeStruct((NC, NP, D), jnp.float32),
    mesh=_MESH,
    scratch_types=[
        pltpu.VMEM((EPT,), jnp.int32),
        pltpu.VMEM((K, D), jnp.float32),
        pltpu.VMEM((K, D), jnp.float32),
        pltpu.VMEM((K, D), jnp.float32),
        pltpu.VMEM((1, K), jnp.int32),
        pltpu.VMEM((1, K), jnp.int32),
        pltpu.VMEM((1, K), jnp.int32),
        pltpu.VMEM((1, K), jnp.int32),
        pltpu.VMEM((1, K), jnp.int32),
        pltpu.VMEM((1, K), jnp.int32),
        pltpu.VMEM_SHARED((NP, D), jnp.float32),
        pltpu.SemaphoreType.DMA,
        pltpu.SemaphoreType.DMA,
        pltpu.SemaphoreType.DMA,
        pltpu.SemaphoreType.DMA,
        pltpu.SemaphoreType.DMA,
        pltpu.SemaphoreType.DMA,
    ],
    compiler_params=_SC_PARAMS,
)


def _tc_emb_body(x_ref, ew_ref, eb_ref, pdeg_ref, h_ref, hp_ref, dinv_ref, dsq_ref):
    h = jnp.dot(x_ref[...], ew_ref[...], preferred_element_type=jnp.float32)
    h = h + eb_ref[...]
    deg = jnp.sum(pdeg_ref[...], axis=0) + 1.0
    dinv = lax.rsqrt(deg)
    h_ref[...] = h
    hp_ref[...] = h * dinv[:, None]
    dinv_ref[...] = dinv
    dsq_ref[...] = 1.0 / deg


_tc_emb = pl.pallas_call(
    _tc_emb_body,
    out_shape=[
        jax.ShapeDtypeStruct((N, D), jnp.float32),
        jax.ShapeDtypeStruct((N, D), jnp.float32),
        jax.ShapeDtypeStruct((N,), jnp.float32),
        jax.ShapeDtypeStruct((N,), jnp.float32),
    ],
)


def _tc_s_body(g_ref, dinv_ref, dsq_ref, s_ref):
    s_ref[...] = dinv_ref[...] * jnp.sum(g_ref[...], axis=0) + dsq_ref[...]


_tc_s = pl.pallas_call(
    _tc_s_body,
    out_shape=jax.ShapeDtypeStruct((N,), jnp.float32),
)


def _tc_step_body(t, dt, p_ref, y_ref, dinv_ref, dsq_ref, s_ref,
                  w1_ref, w0_ref, b_ref, ynew_ref, ypnew_ref):
    dinv = dinv_ref[...]
    y = y_ref[...]
    p = p_ref[0, :N] + p_ref[1, :N]
    agg = dinv[:, None] * p + dsq_ref[...][:, None] * y
    conv = jnp.dot(agg, w1_ref[...], preferred_element_type=jnp.float32)
    conv = conv + (t * s_ref[...])[:, None] * w0_ref[...][None, :] + b_ref[...]
    ynew = y + dt * conv
    ynew_ref[...] = ynew
    ypnew_ref[...] = ynew * dinv[:, None]


def _make_tc_step(t, dt):
    return pl.pallas_call(
        functools.partial(_tc_step_body, t, dt),
        out_shape=[
            jax.ShapeDtypeStruct((N, D), jnp.float32),
            jax.ShapeDtypeStruct((N, D), jnp.float32),
        ],
    )


_TS = np.linspace(0.0, 1.0, NSTEPS)
_TC_STEPS = [_make_tc_step(float(_TS[i - 1]), float(_TS[i] - _TS[i - 1]))
             for i in range(1, NSTEPS)]


def kernel(x, edge_index, emb_W, emb_b, gcn_W, gcn_b):
    src = edge_index[0].astype(jnp.int32)
    dst = edge_index[1].astype(jnp.int32)
    packed = ((src << 14) | dst).reshape(NW, 1, EPT)
    zrows = jnp.zeros((RPT, D), jnp.float32)

    pdeg = _sc_deg(dst).reshape(NW, N)
    h, hp, dinv, dsq = _tc_emb(x, emb_W, emb_b, pdeg)
    g = _sc_g(src, dst, dinv).reshape(NW, N)
    s = _tc_s(g, dinv, dsq)

    w0 = gcn_W[0]
    w1 = gcn_W[1:]
    outs = [h]
    y, yp = h, hp
    for i in range(1, NSTEPS):
        p = _sc_agg(packed, yp, zrows)
        y, yp = _TC_STEPS[i - 1](p, y, dinv, dsq, s, w1, w0, gcn_b)
        outs.append(y)
    return jnp.stack(outs, axis=0)
